# baseline (speedup 1.0000x reference)
import functools

import jax
import jax.numpy as jnp
from jax import lax
from jax.experimental import pallas as pl
from jax.experimental.pallas import tpu as pltpu
from jax.experimental.pallas import tpu_sc as plsc

H_A = 128
H_B = 16
EPS = 1e-5

_NC = 2
_NS = 16
_NW = _NC * _NS
_CHUNK = 128
_IDX_PAD_ROWS = 64


_SUP = 2 * _CHUNK


def _sc_gather(table, idx2, interpret=False):
    total = (idx2.shape[0] - _IDX_PAD_ROWS) * _CHUNK
    d = table.shape[1]
    dt = table.dtype
    unit = 2 * _SUP
    per_sub = total // (_NS * unit)
    mesh = plsc.VectorSubcoreMesh(core_axis_name="c", subcore_axis_name="s")

    @functools.partial(
        pl.kernel,
        out_type=jax.ShapeDtypeStruct((total, d), dt),
        mesh=mesh,
        scratch_types=[
            pltpu.VMEM((per_sub * unit // _CHUNK, _CHUNK), jnp.int32),
            pltpu.VMEM((_SUP, d), dt),
            pltpu.VMEM((_SUP, d), dt),
            pltpu.SemaphoreType.DMA,
            pltpu.SemaphoreType.DMA,
            pltpu.SemaphoreType.DMA,
        ],
        interpret=interpret,
    )
    def gather_k(table_hbm, idx_hbm, out_hbm, idx_v, buf_a, buf_b, gsem,
                 wsem_a, wsem_b):
        c = lax.axis_index("c")
        sub = lax.axis_index("s")
        base = pl.multiple_of(sub * (per_sub * unit), _SUP)
        chunk_base = pl.multiple_of(base // _CHUNK, 8)
        n_rows = per_sub * unit // _CHUNK

        @pl.when(c == 0)
        def _():
            pltpu.sync_copy(idx_hbm.at[pl.ds(chunk_base, n_rows)], idx_v)

            def body(p, carry):
                for buf, wsem, half in ((buf_a, wsem_a, 0),
                                        (buf_b, wsem_b, 1)):
                    sc = p * 2 + half
                    off = sc * _SUP

                    @pl.when(p > 0)
                    def _():
                        pltpu.make_async_copy(
                            buf, out_hbm.at[pl.ds(base, _SUP)], wsem
                        ).wait()

                    for h in range(_SUP // _CHUNK):
                        pltpu.async_copy(
                            table_hbm.at[idx_v.at[sc * (_SUP // _CHUNK) + h]],
                            buf.at[pl.ds(h * _CHUNK, _CHUNK)],
                            gsem,
                        ).wait()
                    pltpu.async_copy(buf,
                                     out_hbm.at[pl.ds(base + off, _SUP)],
                                     wsem)
                return carry

            lax.fori_loop(0, per_sub, body, 0)
            pltpu.make_async_copy(buf_a, out_hbm.at[pl.ds(base, _SUP)],
                                  wsem_a).wait()
            pltpu.make_async_copy(buf_b, out_hbm.at[pl.ds(base, _SUP)],
                                  wsem_b).wait()

    return gather_k(table, idx2)


def _embed_body(nb, nv, idx_ref, emb_ref, xo_ref, xb_ref):
    ids = idx_ref[0, 0, :]
    onehot = (ids[:, None]
              == lax.broadcasted_iota(jnp.int32, (nb, nv), 1)
              ).astype(jnp.float32)
    x = jnp.dot(onehot, emb_ref[...],
                preferred_element_type=jnp.float32,
                precision=jax.lax.Precision.HIGHEST)
    xo_ref[...] = x
    xb_ref[...] = x.astype(jnp.bfloat16)


def _tc_embed(node_attr, emb_table, nb, interpret=False):
    n = node_attr.shape[1]
    grid = n // nb
    nv = (emb_table.shape[0] + 7) // 8 * 8
    emb_pad = jnp.pad(emb_table, ((0, nv - emb_table.shape[0]), (0, 0)))
    idx3 = node_attr.reshape(grid, 1, nb).astype(jnp.int32)
    return pl.pallas_call(
        functools.partial(_embed_body, nb, nv),
        grid=(grid,),
        in_specs=[
            pl.BlockSpec((1, 1, nb), lambda i: (i, 0, 0)),
            pl.BlockSpec((nv, H_A), lambda i: (0, 0)),
        ],
        out_specs=[pl.BlockSpec((nb, H_A), lambda i: (i, 0)),
                   pl.BlockSpec((nb, H_A), lambda i: (i, 0))],
        out_shape=[jax.ShapeDtypeStruct((n, H_A), jnp.float32),
                   jax.ShapeDtypeStruct((n, H_A), jnp.bfloat16)],
        interpret=interpret,
    )(idx3, emb_pad)


def _stats_body(nb, m, x_ref, g_ref, e_ref, wst_ref, wnt_ref, wet_ref,
                sum_ref, sumsq_ref):
    i = pl.program_id(0)
    a = jnp.dot(x_ref[...], wst_ref[...], preferred_element_type=jnp.float32)
    g16 = g_ref[...].astype(jnp.bfloat16)
    bc = jnp.dot(g16, wnt_ref[...], preferred_element_type=jnp.float32)
    bc += jnp.dot(e_ref[...], wet_ref[...], preferred_element_type=jnp.float32)
    gated = bc.reshape(nb, m, 2 * H_A) + a[:, None, :]

    @pl.when(i == 0)
    def _():
        sum_ref[...] = jnp.zeros_like(sum_ref)
        sumsq_ref[...] = jnp.zeros_like(sumsq_ref)

    sum_ref[0, :] += jnp.sum(gated, axis=(0, 1))
    sumsq_ref[0, :] += jnp.sum(gated * gated, axis=(0, 1))


def _main_body(nb, m, x_ref, g_ref, e_ref, wst_ref, wnt_ref, wet_ref,
               scale_ref, shift_ref, summed_ref, s2_ref, ss2_ref):
    i = pl.program_id(0)
    a = jnp.dot(x_ref[...], wst_ref[...], preferred_element_type=jnp.float32)
    g16 = g_ref[...].astype(jnp.bfloat16)
    bc = jnp.dot(g16, wnt_ref[...], preferred_element_type=jnp.float32)
    bc += jnp.dot(e_ref[...], wet_ref[...], preferred_element_type=jnp.float32)
    gated = bc.reshape(nb, m, 2 * H_A) + a[:, None, :]
    gn = gated * scale_ref[0][None, None, :] + shift_ref[0][None, None, :]
    filt = jax.nn.sigmoid(gn[:, :, :H_A])
    core = jnp.maximum(gn[:, :, H_A:], 0.0)
    summed = jnp.sum(filt * core, axis=1)
    summed_ref[...] = summed

    @pl.when(i == 0)
    def _():
        s2_ref[...] = jnp.zeros_like(s2_ref)
        ss2_ref[...] = jnp.zeros_like(ss2_ref)

    s2_ref[0, :] += jnp.sum(summed, axis=0)
    ss2_ref[0, :] += jnp.sum(summed * summed, axis=0)


def _fin_body(round_cs, x_ref, sm_ref, sc2_ref, sh2_ref, sig_ref, tnb_ref,
              xo_ref, xb_ref, cs_ref):
    i = pl.program_id(0)
    xn = jnp.maximum(x_ref[...] + sm_ref[...] * sc2_ref[...] + sh2_ref[...],
                     0.0)
    xn = xn * sig_ref[...] + tnb_ref[...]
    xo_ref[...] = xn
    xb = xn.astype(jnp.bfloat16)
    xb_ref[...] = xb

    @pl.when(i == 0)
    def _():
        cs_ref[...] = jnp.zeros_like(cs_ref)

    if round_cs:
        xn = xb.astype(jnp.float32)
    cs_ref[0, :] += jnp.sum(xn, axis=0)


def _conv_block(x, xb, g, e_flat, W, bias, g1, b1, g2, b2, tw, tb, t0,
                n, m, nb, round_cs=False, interpret=False):
    grid = n // nb
    eb = nb * m
    wst = W[:, :H_A].T.astype(jnp.bfloat16)
    wnt = W[:, H_A:2 * H_A].T.astype(jnp.bfloat16)
    wet = W[:, 2 * H_A:].T

    full = lambda s: pl.BlockSpec(s, lambda i: (0, 0))
    sums, sumsqs = pl.pallas_call(
        functools.partial(_stats_body, nb, m),
        grid=(grid,),
        in_specs=[
            pl.BlockSpec((nb, H_A), lambda i: (i, 0)),
            pl.BlockSpec((eb, H_A), lambda i: (i, 0)),
            pl.BlockSpec((eb, H_B), lambda i: (i, 0)),
            full((H_A, 2 * H_A)),
            full((H_A, 2 * H_A)),
            full((H_B, 2 * H_A)),
        ],
        out_specs=[full((1, 2 * H_A)), full((1, 2 * H_A))],
        out_shape=[jax.ShapeDtypeStruct((1, 2 * H_A), jnp.float32)] * 2,
        interpret=interpret,
    )(xb, g, e_flat, wst, wnt, wet)

    cnt1 = jnp.float32(n * m)
    mean1 = sums[0] / cnt1 + bias
    var1 = sumsqs[0] / cnt1 - (sums[0] / cnt1) ** 2
    scale1 = g1 / jnp.sqrt(var1 + EPS)
    shift1 = b1 + (bias - mean1) * scale1

    summed, s2, ss2 = pl.pallas_call(
        functools.partial(_main_body, nb, m),
        grid=(grid,),
        in_specs=[
            pl.BlockSpec((nb, H_A), lambda i: (i, 0)),
            pl.BlockSpec((eb, H_A), lambda i: (i, 0)),
            pl.BlockSpec((eb, H_B), lambda i: (i, 0)),
            full((H_A, 2 * H_A)),
            full((H_A, 2 * H_A)),
            full((H_B, 2 * H_A)),
            full((1, 2 * H_A)),
            full((1, 2 * H_A)),
        ],
        out_specs=[
            pl.BlockSpec((nb, H_A), lambda i: (i, 0)),
            full((1, H_A)),
            full((1, H_A)),
        ],
        out_shape=[
            jax.ShapeDtypeStruct((n, H_A), jnp.float32),
            jax.ShapeDtypeStruct((1, H_A), jnp.float32),
            jax.ShapeDtypeStruct((1, H_A), jnp.float32),
        ],
        interpret=interpret,
    )(xb, g, e_flat, wst, wnt, wet, scale1[None], shift1[None])

    cnt2 = jnp.float32(n)
    mean2 = s2[0] / cnt2
    var2 = ss2[0] / cnt2 - mean2 ** 2
    scale2 = g2 / jnp.sqrt(var2 + EPS)
    shift2 = b2 - mean2 * scale2
    sigv = jax.nn.sigmoid(t0 * tw[:, 0])
    tnbv = jnp.tanh(t0 * tb[:, 0])

    xo, xob, cs = pl.pallas_call(
        functools.partial(_fin_body, round_cs),
        grid=(grid,),
        in_specs=[
            pl.BlockSpec((nb, H_A), lambda i: (i, 0)),
            pl.BlockSpec((nb, H_A), lambda i: (i, 0)),
            full((1, H_A)),
            full((1, H_A)),
            full((1, H_A)),
            full((1, H_A)),
        ],
        out_specs=[
            pl.BlockSpec((nb, H_A), lambda i: (i, 0)),
            pl.BlockSpec((nb, H_A), lambda i: (i, 0)),
            full((1, H_A)),
        ],
        out_shape=[
            jax.ShapeDtypeStruct((n, H_A), jnp.float32),
            jax.ShapeDtypeStruct((n, H_A), jnp.bfloat16),
            jax.ShapeDtypeStruct((1, H_A), jnp.float32),
        ],
        interpret=interpret,
    )(x, summed, scale2[None], shift2[None], sigv[None], tnbv[None])
    return xo, xob, cs


def _round_bf16(x):
    b = lax.bitcast_convert_type(x, jnp.uint32)
    lsb = (b >> 16) & jnp.uint32(1)
    rounded = (b + jnp.uint32(0x7FFF) + lsb) & jnp.uint32(0xFFFF0000)
    return lax.bitcast_convert_type(rounded, jnp.float32)


def _pad_idx(idx_flat):
    total = idx_flat.shape[0]
    unit = _NW * _CHUNK
    padded = ((total + unit - 1) // unit) * unit
    flat = jnp.concatenate(
        [idx_flat, jnp.zeros((padded - total + _IDX_PAD_ROWS * _CHUNK,),
                             jnp.int32)]
    )
    return flat.reshape(-1, _CHUNK)


def kernel(node_attr, edge_attr, edge_idx, t, emb_table,
           c0_W, c0_b, c0_g1, c0_b1, c0_g2, c0_b2, t0_w, t0_b,
           c1_W, c1_b, c1_g1, c1_b1, c1_g2, c1_b2, t1_w, t1_b,
           eW, eb):
    n = node_attr.shape[1]
    m = edge_idx.shape[2]
    nb = 1000 if n % 1000 == 0 else n
    e_idx = _pad_idx(edge_idx.reshape(-1).astype(jnp.int32))
    e_flat = edge_attr.reshape(-1, H_B)
    t0 = t[0]

    x, xb = _tc_embed(node_attr, emb_table, nb)
    g = _sc_gather(x, e_idx)
    x, xb, _ = _conv_block(x, xb, g, e_flat, c0_W, c0_b, c0_g1, c0_b1, c0_g2,
                           c0_b2, t0_w, t0_b, t0, n, m, nb)
    g = _sc_gather(x, e_idx)
    x, xb, cs = _conv_block(x, xb, g, e_flat, c1_W, c1_b, c1_g1, c1_b1,
                            c1_g2, c1_b2, t1_w, t1_b, t0, n, m, nb,
                            round_cs=True)
    ew16 = _round_bf16(eW[0])
    return jnp.sum(cs[0] * ew16) + jnp.float32(n) * eb[0]


# speedup vs baseline: 1.0969x; 1.0969x over previous
import functools

import jax
import jax.numpy as jnp
from jax import lax
from jax.experimental import pallas as pl
from jax.experimental.pallas import tpu as pltpu
from jax.experimental.pallas import tpu_sc as plsc

H_A = 128
H_B = 16
EPS = 1e-5

_NC = 2
_NS = 16
_NW = _NC * _NS
_CHUNK = 128
_IDX_PAD_ROWS = 64


_NBUF = 4


def _sc_gather(table, idx2, interpret=False):
    total = (idx2.shape[0] - _IDX_PAD_ROWS) * _CHUNK
    d = table.shape[1]
    dt = table.dtype
    n_rows = total // (_NS * _CHUNK)
    n_rounds = n_rows // _NBUF
    mesh = plsc.VectorSubcoreMesh(core_axis_name="c", subcore_axis_name="s")

    @functools.partial(
        pl.kernel,
        out_type=jax.ShapeDtypeStruct((total, d), dt),
        mesh=mesh,
        scratch_types=(
            [pltpu.VMEM((n_rows, _CHUNK), jnp.int32)]
            + [pltpu.VMEM((_CHUNK, d), dt) for _ in range(_NBUF)]
            + [pltpu.SemaphoreType.DMA for _ in range(2 * _NBUF)]
        ),
        interpret=interpret,
    )
    def gather_k(table_hbm, idx_hbm, out_hbm, idx_v, *bufsem):
        bufs = bufsem[:_NBUF]
        gsems = bufsem[_NBUF:2 * _NBUF]
        wsems = bufsem[2 * _NBUF:]
        c = lax.axis_index("c")
        sub = lax.axis_index("s")
        base = pl.multiple_of(sub * (n_rows * _CHUNK), _CHUNK)
        chunk_base = pl.multiple_of(base // _CHUNK, 8)

        @pl.when(c == 0)
        def _():
            pltpu.sync_copy(idx_hbm.at[pl.ds(chunk_base, n_rows)], idx_v)

            def body(r, carry):
                handles = []
                for b in range(_NBUF):
                    @pl.when(r > 0)
                    def _(buf=bufs[b], wsem=wsems[b]):
                        pltpu.make_async_copy(
                            buf, out_hbm.at[pl.ds(base, _CHUNK)], wsem
                        ).wait()

                    handles.append(pltpu.async_copy(
                        table_hbm.at[idx_v.at[r * _NBUF + b]],
                        bufs[b], gsems[b]))
                for b in range(_NBUF):
                    handles[b].wait()
                    pltpu.async_copy(
                        bufs[b],
                        out_hbm.at[pl.ds(base + (r * _NBUF + b) * _CHUNK,
                                         _CHUNK)],
                        wsems[b])
                return carry

            lax.fori_loop(0, n_rounds, body, 0)
            for b in range(_NBUF):
                pltpu.make_async_copy(bufs[b],
                                      out_hbm.at[pl.ds(base, _CHUNK)],
                                      wsems[b]).wait()

    return gather_k(table, idx2)


def _embed_body(nb, nv, idx_ref, emb_ref, xo_ref, xb_ref):
    ids = idx_ref[0, 0, :]
    onehot = (ids[:, None]
              == lax.broadcasted_iota(jnp.int32, (nb, nv), 1)
              ).astype(jnp.float32)
    x = jnp.dot(onehot, emb_ref[...],
                preferred_element_type=jnp.float32,
                precision=jax.lax.Precision.HIGHEST)
    xo_ref[...] = x
    xb_ref[...] = x.astype(jnp.bfloat16)


def _tc_embed(node_attr, emb_table, nb, interpret=False):
    n = node_attr.shape[1]
    grid = n // nb
    nv = (emb_table.shape[0] + 7) // 8 * 8
    emb_pad = jnp.pad(emb_table, ((0, nv - emb_table.shape[0]), (0, 0)))
    idx3 = node_attr.reshape(grid, 1, nb).astype(jnp.int32)
    return pl.pallas_call(
        functools.partial(_embed_body, nb, nv),
        grid=(grid,),
        in_specs=[
            pl.BlockSpec((1, 1, nb), lambda i: (i, 0, 0)),
            pl.BlockSpec((nv, H_A), lambda i: (0, 0)),
        ],
        out_specs=[pl.BlockSpec((nb, H_A), lambda i: (i, 0)),
                   pl.BlockSpec((nb, H_A), lambda i: (i, 0))],
        out_shape=[jax.ShapeDtypeStruct((n, H_A), jnp.float32),
                   jax.ShapeDtypeStruct((n, H_A), jnp.bfloat16)],
        interpret=interpret,
    )(idx3, emb_pad)


def _stats_body(nb, m, x_ref, g_ref, e_ref, wst_ref, wnt_ref, wet_ref,
                sum_ref, sumsq_ref):
    i = pl.program_id(0)
    a = jnp.dot(x_ref[...], wst_ref[...], preferred_element_type=jnp.float32)
    g16 = g_ref[...].astype(jnp.bfloat16)
    bc = jnp.dot(g16, wnt_ref[...], preferred_element_type=jnp.float32)
    bc += jnp.dot(e_ref[...], wet_ref[...], preferred_element_type=jnp.float32)
    gated = bc.reshape(nb, m, 2 * H_A) + a[:, None, :]

    @pl.when(i == 0)
    def _():
        sum_ref[...] = jnp.zeros_like(sum_ref)
        sumsq_ref[...] = jnp.zeros_like(sumsq_ref)

    sum_ref[0, :] += jnp.sum(gated, axis=(0, 1))
    sumsq_ref[0, :] += jnp.sum(gated * gated, axis=(0, 1))


def _main_body(nb, m, x_ref, g_ref, e_ref, wst_ref, wnt_ref, wet_ref,
               scale_ref, shift_ref, summed_ref, s2_ref, ss2_ref):
    i = pl.program_id(0)
    a = jnp.dot(x_ref[...], wst_ref[...], preferred_element_type=jnp.float32)
    g16 = g_ref[...].astype(jnp.bfloat16)
    bc = jnp.dot(g16, wnt_ref[...], preferred_element_type=jnp.float32)
    bc += jnp.dot(e_ref[...], wet_ref[...], preferred_element_type=jnp.float32)
    gated = bc.reshape(nb, m, 2 * H_A) + a[:, None, :]
    gn = gated * scale_ref[0][None, None, :] + shift_ref[0][None, None, :]
    filt = jax.nn.sigmoid(gn[:, :, :H_A])
    core = jnp.maximum(gn[:, :, H_A:], 0.0)
    summed = jnp.sum(filt * core, axis=1)
    summed_ref[...] = summed

    @pl.when(i == 0)
    def _():
        s2_ref[...] = jnp.zeros_like(s2_ref)
        ss2_ref[...] = jnp.zeros_like(ss2_ref)

    s2_ref[0, :] += jnp.sum(summed, axis=0)
    ss2_ref[0, :] += jnp.sum(summed * summed, axis=0)


def _fin_body(round_cs, x_ref, sm_ref, sc2_ref, sh2_ref, sig_ref, tnb_ref,
              xo_ref, xb_ref, cs_ref):
    i = pl.program_id(0)
    xn = jnp.maximum(x_ref[...] + sm_ref[...] * sc2_ref[...] + sh2_ref[...],
                     0.0)
    xn = xn * sig_ref[...] + tnb_ref[...]
    xo_ref[...] = xn
    xb = xn.astype(jnp.bfloat16)
    xb_ref[...] = xb

    @pl.when(i == 0)
    def _():
        cs_ref[...] = jnp.zeros_like(cs_ref)

    if round_cs:
        xn = xb.astype(jnp.float32)
    cs_ref[0, :] += jnp.sum(xn, axis=0)


def _conv_block(x, xb, g, e_flat, W, bias, g1, b1, g2, b2, tw, tb, t0,
                n, m, nb, round_cs=False, interpret=False):
    grid = n // nb
    eb = nb * m
    wst = W[:, :H_A].T.astype(jnp.bfloat16)
    wnt = W[:, H_A:2 * H_A].T.astype(jnp.bfloat16)
    wet = W[:, 2 * H_A:].T

    full = lambda s: pl.BlockSpec(s, lambda i: (0, 0))
    sums, sumsqs = pl.pallas_call(
        functools.partial(_stats_body, nb, m),
        grid=(grid,),
        in_specs=[
            pl.BlockSpec((nb, H_A), lambda i: (i, 0)),
            pl.BlockSpec((eb, H_A), lambda i: (i, 0)),
            pl.BlockSpec((eb, H_B), lambda i: (i, 0)),
            full((H_A, 2 * H_A)),
            full((H_A, 2 * H_A)),
            full((H_B, 2 * H_A)),
        ],
        out_specs=[full((1, 2 * H_A)), full((1, 2 * H_A))],
        out_shape=[jax.ShapeDtypeStruct((1, 2 * H_A), jnp.float32)] * 2,
        interpret=interpret,
    )(xb, g, e_flat, wst, wnt, wet)

    cnt1 = jnp.float32(n * m)
    mean1 = sums[0] / cnt1 + bias
    var1 = sumsqs[0] / cnt1 - (sums[0] / cnt1) ** 2
    scale1 = g1 / jnp.sqrt(var1 + EPS)
    shift1 = b1 + (bias - mean1) * scale1

    summed, s2, ss2 = pl.pallas_call(
        functools.partial(_main_body, nb, m),
        grid=(grid,),
        in_specs=[
            pl.BlockSpec((nb, H_A), lambda i: (i, 0)),
            pl.BlockSpec((eb, H_A), lambda i: (i, 0)),
            pl.BlockSpec((eb, H_B), lambda i: (i, 0)),
            full((H_A, 2 * H_A)),
            full((H_A, 2 * H_A)),
            full((H_B, 2 * H_A)),
            full((1, 2 * H_A)),
            full((1, 2 * H_A)),
        ],
        out_specs=[
            pl.BlockSpec((nb, H_A), lambda i: (i, 0)),
            full((1, H_A)),
            full((1, H_A)),
        ],
        out_shape=[
            jax.ShapeDtypeStruct((n, H_A), jnp.float32),
            jax.ShapeDtypeStruct((1, H_A), jnp.float32),
            jax.ShapeDtypeStruct((1, H_A), jnp.float32),
        ],
        interpret=interpret,
    )(xb, g, e_flat, wst, wnt, wet, scale1[None], shift1[None])

    cnt2 = jnp.float32(n)
    mean2 = s2[0] / cnt2
    var2 = ss2[0] / cnt2 - mean2 ** 2
    scale2 = g2 / jnp.sqrt(var2 + EPS)
    shift2 = b2 - mean2 * scale2
    sigv = jax.nn.sigmoid(t0 * tw[:, 0])
    tnbv = jnp.tanh(t0 * tb[:, 0])

    xo, xob, cs = pl.pallas_call(
        functools.partial(_fin_body, round_cs),
        grid=(grid,),
        in_specs=[
            pl.BlockSpec((nb, H_A), lambda i: (i, 0)),
            pl.BlockSpec((nb, H_A), lambda i: (i, 0)),
            full((1, H_A)),
            full((1, H_A)),
            full((1, H_A)),
            full((1, H_A)),
        ],
        out_specs=[
            pl.BlockSpec((nb, H_A), lambda i: (i, 0)),
            pl.BlockSpec((nb, H_A), lambda i: (i, 0)),
            full((1, H_A)),
        ],
        out_shape=[
            jax.ShapeDtypeStruct((n, H_A), jnp.float32),
            jax.ShapeDtypeStruct((n, H_A), jnp.bfloat16),
            jax.ShapeDtypeStruct((1, H_A), jnp.float32),
        ],
        interpret=interpret,
    )(x, summed, scale2[None], shift2[None], sigv[None], tnbv[None])
    return xo, xob, cs


def _round_bf16(x):
    b = lax.bitcast_convert_type(x, jnp.uint32)
    lsb = (b >> 16) & jnp.uint32(1)
    rounded = (b + jnp.uint32(0x7FFF) + lsb) & jnp.uint32(0xFFFF0000)
    return lax.bitcast_convert_type(rounded, jnp.float32)


def _pad_idx(idx_flat):
    total = idx_flat.shape[0]
    unit = _NW * _CHUNK
    padded = ((total + unit - 1) // unit) * unit
    flat = jnp.concatenate(
        [idx_flat, jnp.zeros((padded - total + _IDX_PAD_ROWS * _CHUNK,),
                             jnp.int32)]
    )
    return flat.reshape(-1, _CHUNK)


def kernel(node_attr, edge_attr, edge_idx, t, emb_table,
           c0_W, c0_b, c0_g1, c0_b1, c0_g2, c0_b2, t0_w, t0_b,
           c1_W, c1_b, c1_g1, c1_b1, c1_g2, c1_b2, t1_w, t1_b,
           eW, eb):
    n = node_attr.shape[1]
    m = edge_idx.shape[2]
    nb = 1000 if n % 1000 == 0 else n
    e_idx = _pad_idx(edge_idx.reshape(-1).astype(jnp.int32))
    e_flat = edge_attr.reshape(-1, H_B)
    t0 = t[0]

    x, xb = _tc_embed(node_attr, emb_table, nb)
    g = _sc_gather(x, e_idx)
    x, xb, _ = _conv_block(x, xb, g, e_flat, c0_W, c0_b, c0_g1, c0_b1, c0_g2,
                           c0_b2, t0_w, t0_b, t0, n, m, nb)
    g = _sc_gather(x, e_idx)
    x, xb, cs = _conv_block(x, xb, g, e_flat, c1_W, c1_b, c1_g1, c1_b1,
                            c1_g2, c1_b2, t1_w, t1_b, t0, n, m, nb,
                            round_cs=True)
    ew16 = _round_bf16(eW[0])
    return jnp.sum(cs[0] * ew16) + jnp.float32(n) * eb[0]


# speedup vs baseline: 1.1260x; 1.0266x over previous
import functools

import jax
import jax.numpy as jnp
from jax import lax
from jax.experimental import pallas as pl
from jax.experimental.pallas import tpu as pltpu
from jax.experimental.pallas import tpu_sc as plsc

H_A = 128
H_B = 16
EPS = 1e-5

_NC = 2
_NS = 16
_NW = _NC * _NS
_CHUNK = 128
_IDX_PAD_ROWS = 64


_NBUF = 4


def _sc_gather(table, idx2, interpret=False):
    total = (idx2.shape[0] - _IDX_PAD_ROWS) * _CHUNK
    d = table.shape[1]
    dt = table.dtype
    per_sub = total // (_NS * _CHUNK)
    u0 = min(per_sub - _NBUF, max(_NBUF, ((per_sub * 7) // 10 + 7) // 8 * 8))
    u1 = per_sub - u0
    mesh = plsc.VectorSubcoreMesh(core_axis_name="c", subcore_axis_name="s")

    @functools.partial(
        pl.kernel,
        out_type=jax.ShapeDtypeStruct((total, d), dt),
        mesh=mesh,
        scratch_types=(
            [pltpu.VMEM((max(u0, u1), _CHUNK), jnp.int32)]
            + [pltpu.VMEM((_CHUNK, d), dt) for _ in range(_NBUF)]
            + [pltpu.SemaphoreType.DMA for _ in range(2 * _NBUF)]
        ),
        interpret=interpret,
    )
    def gather_k(table_hbm, idx_hbm, out_hbm, idx_v, *bufsem):
        bufs = bufsem[:_NBUF]
        gsems = bufsem[_NBUF:2 * _NBUF]
        wsems = bufsem[2 * _NBUF:]
        c = lax.axis_index("c")
        sub = lax.axis_index("s")

        def run(n_chunks, chunk_base):
            chunk_base = pl.multiple_of(chunk_base, 8)
            base = pl.multiple_of(chunk_base * _CHUNK, _CHUNK)
            pltpu.sync_copy(idx_hbm.at[pl.ds(chunk_base, n_chunks)],
                            idx_v.at[pl.ds(0, n_chunks)])

            def body(r, carry):
                handles = []
                for b in range(_NBUF):
                    @pl.when(r > 0)
                    def _(buf=bufs[b], wsem=wsems[b]):
                        pltpu.make_async_copy(
                            buf, out_hbm.at[pl.ds(base, _CHUNK)], wsem
                        ).wait()

                    handles.append(pltpu.async_copy(
                        table_hbm.at[idx_v.at[r * _NBUF + b]],
                        bufs[b], gsems[b]))
                for b in range(_NBUF):
                    handles[b].wait()
                    pltpu.async_copy(
                        bufs[b],
                        out_hbm.at[pl.ds(base + (r * _NBUF + b) * _CHUNK,
                                         _CHUNK)],
                        wsems[b])
                return carry

            lax.fori_loop(0, n_chunks // _NBUF, body, 0)
            for b in range(_NBUF):
                pltpu.make_async_copy(bufs[b],
                                      out_hbm.at[pl.ds(base, _CHUNK)],
                                      wsems[b]).wait()

        @pl.when(c == 0)
        def _():
            run(u0, sub * u0)

        if u1 > 0:
            @pl.when(c == 1)
            def _():
                run(u1, _NS * u0 + sub * u1)

    return gather_k(table, idx2)


def _embed_body(nb, nv, idx_ref, emb_ref, xo_ref, xb_ref):
    ids = idx_ref[0, 0, :]
    onehot = (ids[:, None]
              == lax.broadcasted_iota(jnp.int32, (nb, nv), 1)
              ).astype(jnp.float32)
    x = jnp.dot(onehot, emb_ref[...],
                preferred_element_type=jnp.float32,
                precision=jax.lax.Precision.HIGHEST)
    xo_ref[...] = x
    xb_ref[...] = x.astype(jnp.bfloat16)


def _tc_embed(node_attr, emb_table, nb, interpret=False):
    n = node_attr.shape[1]
    grid = n // nb
    nv = (emb_table.shape[0] + 7) // 8 * 8
    emb_pad = jnp.pad(emb_table, ((0, nv - emb_table.shape[0]), (0, 0)))
    idx3 = node_attr.reshape(grid, 1, nb).astype(jnp.int32)
    return pl.pallas_call(
        functools.partial(_embed_body, nb, nv),
        grid=(grid,),
        in_specs=[
            pl.BlockSpec((1, 1, nb), lambda i: (i, 0, 0)),
            pl.BlockSpec((nv, H_A), lambda i: (0, 0)),
        ],
        out_specs=[pl.BlockSpec((nb, H_A), lambda i: (i, 0)),
                   pl.BlockSpec((nb, H_A), lambda i: (i, 0))],
        out_shape=[jax.ShapeDtypeStruct((n, H_A), jnp.float32),
                   jax.ShapeDtypeStruct((n, H_A), jnp.bfloat16)],
        interpret=interpret,
    )(idx3, emb_pad)


def _stats_body(nb, m, x_ref, g_ref, e_ref, wst_ref, wnt_ref, wet_ref,
                sum_ref, sumsq_ref):
    i = pl.program_id(0)
    a = jnp.dot(x_ref[...], wst_ref[...], preferred_element_type=jnp.float32)
    g16 = g_ref[...].astype(jnp.bfloat16)
    bc = jnp.dot(g16, wnt_ref[...], preferred_element_type=jnp.float32)
    bc += jnp.dot(e_ref[...], wet_ref[...], preferred_element_type=jnp.float32)
    gated = bc.reshape(nb, m, 2 * H_A) + a[:, None, :]

    @pl.when(i == 0)
    def _():
        sum_ref[...] = jnp.zeros_like(sum_ref)
        sumsq_ref[...] = jnp.zeros_like(sumsq_ref)

    sum_ref[0, :] += jnp.sum(gated, axis=(0, 1))
    sumsq_ref[0, :] += jnp.sum(gated * gated, axis=(0, 1))


def _main_body(nb, m, x_ref, g_ref, e_ref, wst_ref, wnt_ref, wet_ref,
               scale_ref, shift_ref, summed_ref, s2_ref, ss2_ref):
    i = pl.program_id(0)
    a = jnp.dot(x_ref[...], wst_ref[...], preferred_element_type=jnp.float32)
    g16 = g_ref[...].astype(jnp.bfloat16)
    bc = jnp.dot(g16, wnt_ref[...], preferred_element_type=jnp.float32)
    bc += jnp.dot(e_ref[...], wet_ref[...], preferred_element_type=jnp.float32)
    gated = bc.reshape(nb, m, 2 * H_A) + a[:, None, :]
    gn = gated * scale_ref[0][None, None, :] + shift_ref[0][None, None, :]
    filt = jax.nn.sigmoid(gn[:, :, :H_A])
    core = jnp.maximum(gn[:, :, H_A:], 0.0)
    summed = jnp.sum(filt * core, axis=1)
    summed_ref[...] = summed

    @pl.when(i == 0)
    def _():
        s2_ref[...] = jnp.zeros_like(s2_ref)
        ss2_ref[...] = jnp.zeros_like(ss2_ref)

    s2_ref[0, :] += jnp.sum(summed, axis=0)
    ss2_ref[0, :] += jnp.sum(summed * summed, axis=0)


def _fin_body(round_cs, x_ref, sm_ref, sc2_ref, sh2_ref, sig_ref, tnb_ref,
              xo_ref, xb_ref, cs_ref):
    i = pl.program_id(0)
    xn = jnp.maximum(x_ref[...] + sm_ref[...] * sc2_ref[...] + sh2_ref[...],
                     0.0)
    xn = xn * sig_ref[...] + tnb_ref[...]
    xo_ref[...] = xn
    xb = xn.astype(jnp.bfloat16)
    xb_ref[...] = xb

    @pl.when(i == 0)
    def _():
        cs_ref[...] = jnp.zeros_like(cs_ref)

    if round_cs:
        xn = xb.astype(jnp.float32)
    cs_ref[0, :] += jnp.sum(xn, axis=0)


def _conv_block(x, xb, g, e_flat, W, bias, g1, b1, g2, b2, tw, tb, t0,
                n, m, nb, round_cs=False, interpret=False):
    grid = n // nb
    eb = nb * m
    wst = W[:, :H_A].T.astype(jnp.bfloat16)
    wnt = W[:, H_A:2 * H_A].T.astype(jnp.bfloat16)
    wet = W[:, 2 * H_A:].T

    full = lambda s: pl.BlockSpec(s, lambda i: (0, 0))
    sums, sumsqs = pl.pallas_call(
        functools.partial(_stats_body, nb, m),
        grid=(grid,),
        in_specs=[
            pl.BlockSpec((nb, H_A), lambda i: (i, 0)),
            pl.BlockSpec((eb, H_A), lambda i: (i, 0)),
            pl.BlockSpec((eb, H_B), lambda i: (i, 0)),
            full((H_A, 2 * H_A)),
            full((H_A, 2 * H_A)),
            full((H_B, 2 * H_A)),
        ],
        out_specs=[full((1, 2 * H_A)), full((1, 2 * H_A))],
        out_shape=[jax.ShapeDtypeStruct((1, 2 * H_A), jnp.float32)] * 2,
        interpret=interpret,
    )(xb, g, e_flat, wst, wnt, wet)

    cnt1 = jnp.float32(n * m)
    mean1 = sums[0] / cnt1 + bias
    var1 = sumsqs[0] / cnt1 - (sums[0] / cnt1) ** 2
    scale1 = g1 / jnp.sqrt(var1 + EPS)
    shift1 = b1 + (bias - mean1) * scale1

    summed, s2, ss2 = pl.pallas_call(
        functools.partial(_main_body, nb, m),
        grid=(grid,),
        in_specs=[
            pl.BlockSpec((nb, H_A), lambda i: (i, 0)),
            pl.BlockSpec((eb, H_A), lambda i: (i, 0)),
            pl.BlockSpec((eb, H_B), lambda i: (i, 0)),
            full((H_A, 2 * H_A)),
            full((H_A, 2 * H_A)),
            full((H_B, 2 * H_A)),
            full((1, 2 * H_A)),
            full((1, 2 * H_A)),
        ],
        out_specs=[
            pl.BlockSpec((nb, H_A), lambda i: (i, 0)),
            full((1, H_A)),
            full((1, H_A)),
        ],
        out_shape=[
            jax.ShapeDtypeStruct((n, H_A), jnp.float32),
            jax.ShapeDtypeStruct((1, H_A), jnp.float32),
            jax.ShapeDtypeStruct((1, H_A), jnp.float32),
        ],
        interpret=interpret,
    )(xb, g, e_flat, wst, wnt, wet, scale1[None], shift1[None])

    cnt2 = jnp.float32(n)
    mean2 = s2[0] / cnt2
    var2 = ss2[0] / cnt2 - mean2 ** 2
    scale2 = g2 / jnp.sqrt(var2 + EPS)
    shift2 = b2 - mean2 * scale2
    sigv = jax.nn.sigmoid(t0 * tw[:, 0])
    tnbv = jnp.tanh(t0 * tb[:, 0])

    xo, xob, cs = pl.pallas_call(
        functools.partial(_fin_body, round_cs),
        grid=(grid,),
        in_specs=[
            pl.BlockSpec((nb, H_A), lambda i: (i, 0)),
            pl.BlockSpec((nb, H_A), lambda i: (i, 0)),
            full((1, H_A)),
            full((1, H_A)),
            full((1, H_A)),
            full((1, H_A)),
        ],
        out_specs=[
            pl.BlockSpec((nb, H_A), lambda i: (i, 0)),
            pl.BlockSpec((nb, H_A), lambda i: (i, 0)),
            full((1, H_A)),
        ],
        out_shape=[
            jax.ShapeDtypeStruct((n, H_A), jnp.float32),
            jax.ShapeDtypeStruct((n, H_A), jnp.bfloat16),
            jax.ShapeDtypeStruct((1, H_A), jnp.float32),
        ],
        interpret=interpret,
    )(x, summed, scale2[None], shift2[None], sigv[None], tnbv[None])
    return xo, xob, cs


def _round_bf16(x):
    b = lax.bitcast_convert_type(x, jnp.uint32)
    lsb = (b >> 16) & jnp.uint32(1)
    rounded = (b + jnp.uint32(0x7FFF) + lsb) & jnp.uint32(0xFFFF0000)
    return lax.bitcast_convert_type(rounded, jnp.float32)


def _pad_idx(idx_flat):
    total = idx_flat.shape[0]
    unit = _NW * _CHUNK
    padded = ((total + unit - 1) // unit) * unit
    flat = jnp.concatenate(
        [idx_flat, jnp.zeros((padded - total + _IDX_PAD_ROWS * _CHUNK,),
                             jnp.int32)]
    )
    return flat.reshape(-1, _CHUNK)


def kernel(node_attr, edge_attr, edge_idx, t, emb_table,
           c0_W, c0_b, c0_g1, c0_b1, c0_g2, c0_b2, t0_w, t0_b,
           c1_W, c1_b, c1_g1, c1_b1, c1_g2, c1_b2, t1_w, t1_b,
           eW, eb):
    n = node_attr.shape[1]
    m = edge_idx.shape[2]
    nb = 1000 if n % 1000 == 0 else n
    e_idx = _pad_idx(edge_idx.reshape(-1).astype(jnp.int32))
    e_flat = edge_attr.reshape(-1, H_B)
    t0 = t[0]

    x, xb = _tc_embed(node_attr, emb_table, nb)
    g = _sc_gather(x, e_idx)
    x, xb, _ = _conv_block(x, xb, g, e_flat, c0_W, c0_b, c0_g1, c0_b1, c0_g2,
                           c0_b2, t0_w, t0_b, t0, n, m, nb)
    g = _sc_gather(x, e_idx)
    x, xb, cs = _conv_block(x, xb, g, e_flat, c1_W, c1_b, c1_g1, c1_b1,
                            c1_g2, c1_b2, t1_w, t1_b, t0, n, m, nb,
                            round_cs=True)
    ew16 = _round_bf16(eW[0])
    return jnp.sum(cs[0] * ew16) + jnp.float32(n) * eb[0]
